# baseline (speedup 1.0000x reference)
import functools

import jax
import jax.numpy as jnp
from jax.experimental import pallas as pl
from jax.experimental.pallas import tpu as pltpu

SLOTS = 65536
D = 256
LANES = 128
ROWS = SLOTS // LANES
BLK_R = 16
NJ = ROWS // BLK_R


def _body(c_ref, keys_ref, vals_ref, alpha_ref, vhat_ref,
          sims_s, acc_v, m_s):
    p = pl.program_id(0)
    j = pl.program_id(1)

    @pl.when(p == 0)
    def _keys_phase():
        kb = keys_ref[...].astype(jnp.bfloat16).astype(jnp.float32)
        cb = c_ref[...].astype(jnp.bfloat16).astype(jnp.float32)
        sims = jnp.sum(kb * cb, axis=2)
        sims_s[pl.ds(j * BLK_R, BLK_R), :] = sims
        bmax = jnp.max(sims)
        prev = jnp.where(j == 0, -jnp.inf, m_s[0])
        m_s[0] = jnp.maximum(prev, bmax)

    @pl.when((p == 1) & (j == 0))
    def _softmax_phase():
        e = jnp.exp(sims_s[...] - m_s[0])
        sims_s[...] = e * (1.0 / jnp.sum(e))

    @pl.when(p == 1)
    def _vals_phase():
        a = sims_s[pl.ds(j * BLK_R, BLK_R), :]
        alpha_ref[...] = a
        ab = a.astype(jnp.bfloat16).astype(jnp.float32)
        vb = vals_ref[...].astype(jnp.bfloat16).astype(jnp.float32)
        part = jnp.sum(ab[:, :, None] * vb, axis=(0, 1))
        prev = jnp.where(j == 0, jnp.zeros((1, D), jnp.float32), acc_v[...])
        acc_v[...] = prev + part.reshape(1, D)

        @pl.when(j == NJ - 1)
        def _emit():
            vhat_ref[...] = acc_v[...]


@jax.jit
def kernel(c, keys, vals):
    keys3 = keys.reshape(ROWS, LANES, D)
    vals3 = vals.reshape(ROWS, LANES, D)
    c3 = c.reshape(1, 1, D)

    grid = (2, NJ)
    alpha2d, vhat2d = pl.pallas_call(
        _body,
        grid=grid,
        in_specs=[
            pl.BlockSpec((1, 1, D), lambda p, j: (0, 0, 0)),
            pl.BlockSpec((BLK_R, LANES, D),
                         lambda p, j: (jnp.where(p == 0, j, 0), 0, 0)),
            pl.BlockSpec((BLK_R, LANES, D),
                         lambda p, j: (jnp.where(p == 1, j, 0), 0, 0)),
        ],
        out_specs=[
            pl.BlockSpec((BLK_R, LANES),
                         lambda p, j: (jnp.where(p == 1, j, 0), 0)),
            pl.BlockSpec((1, D), lambda p, j: (0, 0)),
        ],
        out_shape=[
            jax.ShapeDtypeStruct((ROWS, LANES), jnp.float32),
            jax.ShapeDtypeStruct((1, D), jnp.float32),
        ],
        scratch_shapes=[
            pltpu.VMEM((ROWS, LANES), jnp.float32),
            pltpu.VMEM((1, D), jnp.float32),
            pltpu.SMEM((1,), jnp.float32),
        ],
        compiler_params=pltpu.CompilerParams(
            dimension_semantics=("arbitrary", "arbitrary"),
        ),
    )(c3, keys3, vals3)
    return (vhat2d.reshape(D), alpha2d.reshape(SLOTS))
